# Initial kernel scaffold; baseline (speedup 1.0000x reference)
import functools

import jax
import jax.numpy as jnp
from jax import lax
from jax.experimental import pallas as pl
from jax.experimental.pallas import tpu as pltpu
import jax.experimental.pallas.tpu_sc as plsc

NC, NS, L = 2, 16, 16
NW = NC * NS
EK = 128


def _norm_body(x_ref, o_ref):
    x = x_ref[...]
    ss = jnp.sum(x * x, axis=1, keepdims=True)
    o_ref[...] = x / jnp.sqrt(ss)


def _normalize(x, block_rows=200):
    n, c = x.shape
    grid = n // block_rows
    return pl.pallas_call(
        _norm_body,
        grid=(grid,),
        in_specs=[pl.BlockSpec((block_rows, c), lambda i: (i, 0))],
        out_specs=pl.BlockSpec((block_rows, c), lambda i: (i, 0)),
        out_shape=jax.ShapeDtypeStruct((n, c), x.dtype),
    )(x)


def _table_body(ps_ref, pd_ref, po_ref, ro_ref, wq_ref, bq_ref, wk_ref,
                bk_ref, wv_ref, bv_ref, wo_ref, bo_ref, o_ref):
    c = ps_ref.shape[-1]
    neg = jnp.full((1, c), -1.0, dtype=jnp.float32)
    ro = ro_ref[...].reshape(1, c)
    scale = 1.0 / jnp.sqrt(jnp.asarray(c, jnp.float32))
    rows = []
    for k in range(4):
        simf = bool(k & 1)
        degf = bool(k & 2)
        otherf = not (simf or degf)
        slot_sim = ps_ref[...] if simf else neg
        slot_deg = pd_ref[...] if degf else neg
        slot_other = po_ref[...] if otherf else neg
        rec = jnp.concatenate(
            [ro, slot_sim, slot_deg, slot_other,
             jnp.zeros((4, c), jnp.float32)], axis=0)
        pad = jnp.all(rec == -1.0, axis=-1) | (lax.iota(jnp.int32, 8) >= 4)
        dn = (((1,), (1,)), ((), ()))
        q = lax.dot_general(rec, wq_ref[...], dn) + bq_ref[...]
        kk = lax.dot_general(rec, wk_ref[...], dn) + bk_ref[...]
        v = lax.dot_general(rec, wv_ref[...], dn) + bv_ref[...]
        scores = lax.dot_general(q, kk, dn) * scale
        scores = jnp.where(pad[None, :], -1e30, scores)
        m = jnp.max(scores, axis=-1, keepdims=True)
        ex = jnp.exp(scores - m)
        attn = ex / jnp.sum(ex, axis=-1, keepdims=True)
        av = jnp.dot(attn, v)
        out = lax.dot_general(av, wo_ref[...], dn) + bo_ref[...]
        rows.append(out[0:1, :])
    rows.append(jnp.zeros((4, c), jnp.float32))
    o_ref[...] = jnp.concatenate(rows, axis=0)


def _make_table(prompt_sim, prompt_deg, prompt_other, readout,
                Wq, bq, Wk, bk, Wv, bv, Wo, bo):
    c = prompt_sim.shape[-1]
    return pl.pallas_call(
        _table_body,
        out_shape=jax.ShapeDtypeStruct((8, c), jnp.float32),
    )(prompt_sim, prompt_deg, prompt_other, readout,
      Wq, bq, Wk, bk, Wv, bv, Wo, bo)


def _edge_sc(x_norm, edge_index):
    n, c = x_norm.shape
    e_total = edge_index.shape[1]
    nchunks = e_total // EK
    base_trips = nchunks // NW
    extra = nchunks - base_trips * NW
    zr = n // NS
    mesh = plsc.VectorSubcoreMesh(core_axis_name="c", subcore_axis_name="s")

    @functools.partial(
        pl.kernel,
        out_type=jax.ShapeDtypeStruct((NC, n, L), jnp.float32),
        mesh=mesh,
        scratch_types=[
            pltpu.VMEM((EK,), jnp.int32),
            pltpu.VMEM((EK,), jnp.int32),
            pltpu.VMEM((EK, c), jnp.float32),
            pltpu.VMEM((EK, c), jnp.float32),
            pltpu.VMEM((EK, L), jnp.float32),
            pltpu.VMEM((zr, L), jnp.float32),
            pltpu.VMEM_SHARED((n, L), jnp.float32),
            pltpu.SemaphoreType.DMA,
            pltpu.SemaphoreType.DMA,
        ],
    )
    def edge_kernel(xn_hbm, edges_hbm, out_hbm, ridx_v, cidx_v, rrows_v,
                    crows_v, upd_v, zero_v, acc_sh, sem1, sem2):
        cid = lax.axis_index("c")
        sid = lax.axis_index("s")
        wid = cid * NS + sid
        lanes = lax.iota(jnp.int32, L)
        zvec = jnp.zeros((L,), jnp.float32)

        def zero_zbuf(i, _):
            zero_v[i, :] = zvec
            return 0

        lax.fori_loop(0, zr, zero_zbuf, 0)
        pltpu.sync_copy(zero_v, acc_sh.at[pl.ds(sid * zr, zr)])

        def zero_upd(i, _):
            upd_v[i, :] = zvec
            return 0

        lax.fori_loop(0, EK, zero_upd, 0)
        ones_f = jnp.ones((L,), jnp.float32)
        ones_i = jnp.ones((L,), jnp.int32)
        zeros_i = jnp.zeros((L,), jnp.int32)
        for g in range(EK // L):
            plsc.store_scatter(upd_v, [g * L + lanes, ones_i], ones_f)
        plsc.subcore_barrier()

        trips = base_trips + jnp.where(wid < extra, 1, 0)

        def chunk_body(i, _):
            ebase = (i * NW + wid) * EK
            pltpu.sync_copy(edges_hbm.at[0, pl.ds(ebase, EK)], ridx_v)
            pltpu.sync_copy(edges_hbm.at[1, pl.ds(ebase, EK)], cidx_v)
            cp1 = pltpu.async_copy(xn_hbm.at[ridx_v], rrows_v, sem1)
            cp2 = pltpu.async_copy(xn_hbm.at[cidx_v], crows_v, sem2)
            cp1.wait()
            cp2.wait()
            for g in range(EK // L):
                lane_rows = g * L + lanes

                def jbody(j, acc):
                    jv = jnp.full((L,), j, jnp.int32)
                    a = plsc.load_gather(rrows_v, [lane_rows, jv])
                    b = plsc.load_gather(crows_v, [lane_rows, jv])
                    return acc + a * b

                acc = lax.fori_loop(0, c, jbody, zvec)
                plsc.store_scatter(upd_v, [lane_rows, zeros_i], acc)
            pltpu.sync_copy(upd_v, acc_sh.at[cidx_v], add=True)
            return 0

        lax.fori_loop(0, trips, chunk_body, 0)
        plsc.subcore_barrier()

        @pl.when(sid == 0)
        def _():
            pltpu.sync_copy(acc_sh, out_hbm.at[cid])

    return edge_kernel(x_norm, edge_index)


def _final_body(x_ref, part_ref, tbl_ref, o_ref):
    part = part_ref[...]
    summed = part[0] + part[1]
    csum = summed[:, 0:1]
    deg = summed[:, 1:2]
    csim = csum / deg
    sim_mask = csim <= 0.2
    deg_mask = deg <= 3.0
    tbl = tbl_ref[...]
    acc = x_ref[...]
    for k in range(4):
        m = (sim_mask == bool(k & 1)) & (deg_mask == bool(k & 2))
        acc = acc + jnp.where(m, tbl[k:k + 1, :], 0.0)
    o_ref[...] = acc


def _finalize(x, partials, table, block_rows=200):
    n, c = x.shape
    grid = n // block_rows
    return pl.pallas_call(
        _final_body,
        grid=(grid,),
        in_specs=[
            pl.BlockSpec((block_rows, c), lambda i: (i, 0)),
            pl.BlockSpec((NC, block_rows, L), lambda i: (0, i, 0)),
            pl.BlockSpec((8, c), lambda i: (0, 0)),
        ],
        out_specs=pl.BlockSpec((block_rows, c), lambda i: (i, 0)),
        out_shape=jax.ShapeDtypeStruct((n, c), x.dtype),
    )(x, partials, table)


@jax.jit
def kernel(x, edge_index, prompt_sim, prompt_deg, prompt_other, readout,
           Wq, bq, Wk, bk, Wv, bv, Wo, bo):
    x_norm = _normalize(x)
    table = _make_table(prompt_sim, prompt_deg, prompt_other, readout,
                        Wq, bq, Wk, bk, Wv, bv, Wo, bo)
    partials = _edge_sc(x_norm, edge_index)
    return _finalize(x, partials, table)


# speedup vs baseline: 1.5901x; 1.5901x over previous
import functools

import jax
import jax.numpy as jnp
from jax import lax
from jax.experimental import pallas as pl
from jax.experimental.pallas import tpu as pltpu
import jax.experimental.pallas.tpu_sc as plsc

NC, NS, L = 2, 16, 16
NW = NC * NS
EK = 128


def _norm_body(x_ref, o_ref):
    x = x_ref[...]
    ss = jnp.sum(x * x, axis=1, keepdims=True)
    o_ref[...] = x / jnp.sqrt(ss)


def _normalize(x, block_rows=2000):
    n, c = x.shape
    grid = pl.cdiv(n, block_rows)
    return pl.pallas_call(
        _norm_body,
        grid=(grid,),
        in_specs=[pl.BlockSpec((block_rows, c), lambda i: (i, 0))],
        out_specs=pl.BlockSpec((block_rows, c), lambda i: (i, 0)),
        out_shape=jax.ShapeDtypeStruct((n, c), x.dtype),
    )(x)


def _table_body(ps_ref, pd_ref, po_ref, ro_ref, wq_ref, bq_ref, wk_ref,
                bk_ref, wv_ref, bv_ref, wo_ref, bo_ref, o_ref):
    c = ps_ref.shape[-1]
    neg = jnp.full((1, c), -1.0, dtype=jnp.float32)
    ro = ro_ref[...].reshape(1, c)
    scale = 1.0 / jnp.sqrt(jnp.asarray(c, jnp.float32))
    rows = []
    for k in range(4):
        simf = bool(k & 1)
        degf = bool(k & 2)
        otherf = not (simf or degf)
        slot_sim = ps_ref[...] if simf else neg
        slot_deg = pd_ref[...] if degf else neg
        slot_other = po_ref[...] if otherf else neg
        rec = jnp.concatenate(
            [ro, slot_sim, slot_deg, slot_other,
             jnp.zeros((4, c), jnp.float32)], axis=0)
        pad = jnp.all(rec == -1.0, axis=-1) | (lax.iota(jnp.int32, 8) >= 4)
        dn = (((1,), (1,)), ((), ()))
        q = lax.dot_general(rec, wq_ref[...], dn) + bq_ref[...]
        kk = lax.dot_general(rec, wk_ref[...], dn) + bk_ref[...]
        v = lax.dot_general(rec, wv_ref[...], dn) + bv_ref[...]
        scores = lax.dot_general(q, kk, dn) * scale
        scores = jnp.where(pad[None, :], -1e30, scores)
        m = jnp.max(scores, axis=-1, keepdims=True)
        ex = jnp.exp(scores - m)
        attn = ex / jnp.sum(ex, axis=-1, keepdims=True)
        av = jnp.dot(attn, v)
        out = lax.dot_general(av, wo_ref[...], dn) + bo_ref[...]
        rows.append(out[0:1, :])
    rows.append(jnp.zeros((4, c), jnp.float32))
    o_ref[...] = jnp.concatenate(rows, axis=0)


def _make_table(prompt_sim, prompt_deg, prompt_other, readout,
                Wq, bq, Wk, bk, Wv, bv, Wo, bo):
    c = prompt_sim.shape[-1]
    return pl.pallas_call(
        _table_body,
        out_shape=jax.ShapeDtypeStruct((8, c), jnp.float32),
    )(prompt_sim, prompt_deg, prompt_other, readout,
      Wq, bq, Wk, bk, Wv, bv, Wo, bo)


def _edge_sc(x_norm, edge_flat, n, c, e_total):
    nchunks = e_total // EK
    base_trips = nchunks // NW
    extra = nchunks - base_trips * NW
    mesh = plsc.VectorSubcoreMesh(core_axis_name="c", subcore_axis_name="s")

    @functools.partial(
        pl.kernel,
        out_type=jax.ShapeDtypeStruct((2 * NW, n), jnp.float32),
        mesh=mesh,
        compiler_params=pltpu.CompilerParams(needs_layout_passes=False),
        scratch_types=[
            pltpu.VMEM((EK,), jnp.int32),
            pltpu.VMEM((EK,), jnp.int32),
            pltpu.VMEM((EK, c), jnp.float32),
            pltpu.VMEM((EK, c), jnp.float32),
            pltpu.VMEM((n,), jnp.float32),
            pltpu.VMEM((n,), jnp.float32),
            pltpu.SemaphoreType.DMA,
            pltpu.SemaphoreType.DMA,
        ],
    )
    def edge_kernel(xn_hbm, edges_hbm, out_hbm, ridx_v, cidx_v, rrows_v,
                    crows_v, c_v, deg_v, sem1, sem2):
        cid = lax.axis_index("c")
        sid = lax.axis_index("s")
        wid = cid * NS + sid
        lanes = lax.iota(jnp.int32, L)
        zvec = jnp.zeros((L,), jnp.float32)
        ones_f = jnp.ones((L,), jnp.float32)

        def zero_acc(i, _):
            c_v[pl.ds(i * L, L)] = zvec
            deg_v[pl.ds(i * L, L)] = zvec
            return 0

        lax.fori_loop(0, n // L, zero_acc, 0)

        trips = base_trips + jnp.where(wid < extra, 1, 0)

        def chunk_body(i, _):
            ebase = (i * NW + wid) * EK
            pltpu.sync_copy(edges_hbm.at[pl.ds(ebase, EK)], ridx_v)
            pltpu.sync_copy(edges_hbm.at[pl.ds(e_total + ebase, EK)], cidx_v)
            cp1 = pltpu.async_copy(xn_hbm.at[ridx_v], rrows_v, sem1)
            cp2 = pltpu.async_copy(xn_hbm.at[cidx_v], crows_v, sem2)
            cp1.wait()
            cp2.wait()
            for g in range(EK // L):
                lane_rows = g * L + lanes

                def jbody(j, acc):
                    jv = jnp.full((L,), j, jnp.int32)
                    a = plsc.load_gather(rrows_v, [lane_rows, jv])
                    b = plsc.load_gather(crows_v, [lane_rows, jv])
                    return acc + a * b

                acc = lax.fori_loop(0, c, jbody, zvec)
                cols = cidx_v[pl.ds(g * L, L)]
                occ, _ = plsc.scan_count(cols)
                mx = lax.reduce_max(occ, (0,))

                def peel(r, _):
                    sel = occ == r
                    plsc.addupdate_scatter(c_v, [cols], acc, mask=sel)
                    plsc.addupdate_scatter(deg_v, [cols], ones_f, mask=sel)
                    return 0

                lax.fori_loop(0, mx + 1, peel, 0)
            return 0

        lax.fori_loop(0, trips, chunk_body, 0)
        pltpu.sync_copy(c_v, out_hbm.at[2 * wid])
        pltpu.sync_copy(deg_v, out_hbm.at[2 * wid + 1])

    return edge_kernel(x_norm, edge_flat)


def _final_body(x_ref, part_ref, tbl_ref, o_ref):
    part = part_ref[...]
    summed = jnp.sum(part.reshape(NW, 2, part.shape[-1]), axis=0)
    cd = jnp.transpose(summed)
    csum = cd[:, 0:1]
    deg = cd[:, 1:2]
    csim = csum / deg
    sim_mask = csim <= 0.2
    deg_mask = deg <= 3.0
    tbl = tbl_ref[...]
    acc = x_ref[...]
    for k in range(4):
        m = (sim_mask == bool(k & 1)) & (deg_mask == bool(k & 2))
        acc = acc + jnp.where(m, tbl[k:k + 1, :], 0.0)
    o_ref[...] = acc


def _finalize(x, partials, table, block_rows=2048):
    n, c = x.shape
    grid = pl.cdiv(n, block_rows)
    return pl.pallas_call(
        _final_body,
        grid=(grid,),
        in_specs=[
            pl.BlockSpec((block_rows, c), lambda i: (i, 0)),
            pl.BlockSpec((2 * NW, block_rows), lambda i: (0, i)),
            pl.BlockSpec((8, c), lambda i: (0, 0)),
        ],
        out_specs=pl.BlockSpec((block_rows, c), lambda i: (i, 0)),
        out_shape=jax.ShapeDtypeStruct((n, c), x.dtype),
    )(x, partials, table)


@jax.jit
def kernel(x, edge_index, prompt_sim, prompt_deg, prompt_other, readout,
           Wq, bq, Wk, bk, Wv, bv, Wo, bo):
    n, c = x.shape
    e_total = edge_index.shape[1]
    x_norm = _normalize(x)
    table = _make_table(prompt_sim, prompt_deg, prompt_other, readout,
                        Wq, bq, Wk, bk, Wv, bv, Wo, bo)
    partials = _edge_sc(x_norm, edge_index.reshape(2 * e_total), n, c,
                        e_total)
    return _finalize(x, partials, table)
